# baseline (speedup 1.0000x reference)
import functools

import jax
import jax.numpy as jnp
from jax import lax
from jax.experimental import pallas as pl
from jax.experimental.pallas import tpu as pltpu
from jax.experimental.pallas import tpu_sc as plsc

B, T, D = 4, 4096, 2048
E = 16
N = B * T
NW = 32
LANES = 16
CHUNKS = (4096, 4096, 4096, 4096)


def _logits_body(x_ref, w_ref, b_ref, out_ref):
    acc = lax.dot_general(
        w_ref[...], x_ref[...],
        dimension_numbers=(((1,), (1,)), ((), ())),
        preferred_element_type=jnp.float32,
    )
    out_ref[0] = acc + b_ref[...]


def _compute_logits_t(x2d, W, b2d, base_tok, chunk):
    per_w = chunk // NW
    base_blk = base_tok // per_w
    return pl.pallas_call(
        _logits_body,
        grid=(NW,),
        in_specs=[
            pl.BlockSpec((per_w, D), lambda i: (base_blk + i, 0)),
            pl.BlockSpec((E, D), lambda i: (0, 0)),
            pl.BlockSpec((E, 1), lambda i: (0, 0)),
        ],
        out_specs=pl.BlockSpec((1, E, per_w), lambda i: (i, 0, 0)),
        out_shape=jax.ShapeDtypeStruct((NW, E, per_w), jnp.float32),
    )(x2d, W, b2d)


def _route_body(per_w, lg_hbm, v1_hbm, v2_hbm, i1_hbm, i2_hbm,
                lg_v, v1_v, v2_v, i1_v, i2_v):
    wid = lax.axis_index("s") * 2 + lax.axis_index("c")
    pltpu.sync_copy(lg_hbm.at[wid], lg_v)

    def group(j, carry):
        base = j * LANES
        vecs = [lg_v[e, pl.ds(base, LANES)] for e in range(E)]
        m = vecs[0]
        for e in range(1, E):
            m = jnp.maximum(m, vecs[e])
        s = jnp.exp(vecs[0] - m)
        for e in range(1, E):
            s = s + jnp.exp(vecs[e] - m)
        max1 = vecs[0]
        idx1 = jnp.zeros((LANES,), jnp.int32)
        max2 = jnp.full((LANES,), -jnp.inf, jnp.float32)
        idx2 = jnp.zeros((LANES,), jnp.int32)
        for e in range(1, E):
            ve = vecs[e]
            eidx = jnp.full((LANES,), e, jnp.int32)
            gt1 = ve > max1
            gt2 = ve > max2
            max2 = jnp.where(gt1, max1, jnp.where(gt2, ve, max2))
            idx2 = jnp.where(gt1, idx1, jnp.where(gt2, eidx, idx2))
            max1 = jnp.where(gt1, ve, max1)
            idx1 = jnp.where(gt1, eidx, idx1)
        inv = 1.0 / s
        v1_v[pl.ds(base, LANES)] = inv
        v2_v[pl.ds(base, LANES)] = jnp.exp(max2 - m) * inv
        i1_v[pl.ds(base, LANES)] = idx1
        i2_v[pl.ds(base, LANES)] = idx2
        return carry

    lax.fori_loop(0, per_w // LANES, group, None)

    out_base = wid * per_w
    pltpu.sync_copy(v1_v, v1_hbm.at[pl.ds(out_base, per_w)])
    pltpu.sync_copy(v2_v, v2_hbm.at[pl.ds(out_base, per_w)])
    pltpu.sync_copy(i1_v, i1_hbm.at[pl.ds(out_base, per_w)])
    pltpu.sync_copy(i2_v, i2_hbm.at[pl.ds(out_base, per_w)])


def _route_topk(logits_t, chunk):
    per_w = chunk // NW
    mesh = plsc.VectorSubcoreMesh(core_axis_name="c", subcore_axis_name="s")
    f = pl.kernel(
        functools.partial(_route_body, per_w),
        out_type=[
            jax.ShapeDtypeStruct((chunk,), jnp.float32),
            jax.ShapeDtypeStruct((chunk,), jnp.float32),
            jax.ShapeDtypeStruct((chunk,), jnp.int32),
            jax.ShapeDtypeStruct((chunk,), jnp.int32),
        ],
        mesh=mesh,
        scratch_types=[
            pltpu.VMEM((E, per_w), jnp.float32),
            pltpu.VMEM((per_w,), jnp.float32),
            pltpu.VMEM((per_w,), jnp.float32),
            pltpu.VMEM((per_w,), jnp.int32),
            pltpu.VMEM((per_w,), jnp.int32),
        ],
    )
    return f(logits_t)


def _fused_body(x_ref, w_ref, b_ref, iv_ref, vv_ref):
    lgt = lax.dot_general(
        w_ref[...], x_ref[...],
        dimension_numbers=(((1,), (1,)), ((), ())),
        preferred_element_type=jnp.float32,
    ) + b_ref[...]
    tb = lgt.shape[1]
    m = jnp.max(lgt, axis=0, keepdims=True)
    s = jnp.sum(jnp.exp(lgt - m), axis=0, keepdims=True)
    iota = lax.broadcasted_iota(jnp.int32, (E, tb), 0)
    idx1 = jnp.min(jnp.where(lgt == m, iota, E), axis=0, keepdims=True)
    lgt2 = jnp.where(iota == idx1, -jnp.inf, lgt)
    m2 = jnp.max(lgt2, axis=0, keepdims=True)
    idx2 = jnp.min(jnp.where(lgt2 == m2, iota, E), axis=0, keepdims=True)
    inv = 1.0 / s
    iv_ref[...] = jnp.transpose(jnp.concatenate([idx1, idx2], axis=0), (1, 0))
    vv_ref[...] = jnp.transpose(
        jnp.concatenate([inv, jnp.exp(m2 - m) * inv], axis=0), (1, 0))


def _fused_tc(x2d, W, b2d, tblk):
    nblk = N // tblk
    return pl.pallas_call(
        _fused_body,
        grid=(nblk,),
        in_specs=[
            pl.BlockSpec((tblk, D), lambda i: (i, 0)),
            pl.BlockSpec((E, D), lambda i: (0, 0)),
            pl.BlockSpec((E, 1), lambda i: (0, 0)),
        ],
        out_specs=[
            pl.BlockSpec((tblk, 2), lambda i: (i, 0)),
            pl.BlockSpec((tblk, 2), lambda i: (i, 0)),
        ],
        out_shape=[
            jax.ShapeDtypeStruct((N, 2), jnp.int32),
            jax.ShapeDtypeStruct((N, 2), jnp.float32),
        ],
    )(x2d, W, b2d)


def kernel(x, W, b):
    x2d = x.reshape(N, D)
    b2d = b.reshape(E, 1)
    iv, vv = _fused_tc(x2d, W, b2d, 1024)
    return (iv.reshape(B, T, 2), vv.reshape(B, T, 2))


# speedup vs baseline: 1.4005x; 1.4005x over previous
import functools

import jax
import jax.numpy as jnp
from jax import lax
from jax.experimental import pallas as pl
from jax.experimental.pallas import tpu as pltpu
from jax.experimental.pallas import tpu_sc as plsc

B, T, D = 4, 4096, 2048
E = 16
N = B * T
NW = 32
LANES = 16
CHUNKS = (4096, 4096, 4096, 4096)


def _logits_body(x_ref, w_ref, b_ref, out_ref):
    acc = lax.dot_general(
        w_ref[...], x_ref[...],
        dimension_numbers=(((1,), (1,)), ((), ())),
        preferred_element_type=jnp.float32,
    )
    out_ref[0] = acc + b_ref[...]


def _compute_logits_t(x2d, W, b2d, base_tok, chunk):
    per_w = chunk // NW
    base_blk = base_tok // per_w
    return pl.pallas_call(
        _logits_body,
        grid=(NW,),
        in_specs=[
            pl.BlockSpec((per_w, D), lambda i: (base_blk + i, 0)),
            pl.BlockSpec((E, D), lambda i: (0, 0)),
            pl.BlockSpec((E, 1), lambda i: (0, 0)),
        ],
        out_specs=pl.BlockSpec((1, E, per_w), lambda i: (i, 0, 0)),
        out_shape=jax.ShapeDtypeStruct((NW, E, per_w), jnp.float32),
    )(x2d, W, b2d)


def _route_body(per_w, lg_hbm, v1_hbm, v2_hbm, i1_hbm, i2_hbm,
                lg_v, v1_v, v2_v, i1_v, i2_v):
    wid = lax.axis_index("s") * 2 + lax.axis_index("c")
    pltpu.sync_copy(lg_hbm.at[wid], lg_v)

    def group(j, carry):
        base = j * LANES
        vecs = [lg_v[e, pl.ds(base, LANES)] for e in range(E)]
        m = vecs[0]
        for e in range(1, E):
            m = jnp.maximum(m, vecs[e])
        s = jnp.exp(vecs[0] - m)
        for e in range(1, E):
            s = s + jnp.exp(vecs[e] - m)
        max1 = vecs[0]
        idx1 = jnp.zeros((LANES,), jnp.int32)
        max2 = jnp.full((LANES,), -jnp.inf, jnp.float32)
        idx2 = jnp.zeros((LANES,), jnp.int32)
        for e in range(1, E):
            ve = vecs[e]
            eidx = jnp.full((LANES,), e, jnp.int32)
            gt1 = ve > max1
            gt2 = ve > max2
            max2 = jnp.where(gt1, max1, jnp.where(gt2, ve, max2))
            idx2 = jnp.where(gt1, idx1, jnp.where(gt2, eidx, idx2))
            max1 = jnp.where(gt1, ve, max1)
            idx1 = jnp.where(gt1, eidx, idx1)
        inv = 1.0 / s
        v1_v[pl.ds(base, LANES)] = inv
        v2_v[pl.ds(base, LANES)] = jnp.exp(max2 - m) * inv
        i1_v[pl.ds(base, LANES)] = idx1
        i2_v[pl.ds(base, LANES)] = idx2
        return carry

    lax.fori_loop(0, per_w // LANES, group, None)

    out_base = wid * per_w
    pltpu.sync_copy(v1_v, v1_hbm.at[pl.ds(out_base, per_w)])
    pltpu.sync_copy(v2_v, v2_hbm.at[pl.ds(out_base, per_w)])
    pltpu.sync_copy(i1_v, i1_hbm.at[pl.ds(out_base, per_w)])
    pltpu.sync_copy(i2_v, i2_hbm.at[pl.ds(out_base, per_w)])


def _route_topk(logits_t, chunk):
    per_w = chunk // NW
    mesh = plsc.VectorSubcoreMesh(core_axis_name="c", subcore_axis_name="s")
    f = pl.kernel(
        functools.partial(_route_body, per_w),
        out_type=[
            jax.ShapeDtypeStruct((chunk,), jnp.float32),
            jax.ShapeDtypeStruct((chunk,), jnp.float32),
            jax.ShapeDtypeStruct((chunk,), jnp.int32),
            jax.ShapeDtypeStruct((chunk,), jnp.int32),
        ],
        mesh=mesh,
        scratch_types=[
            pltpu.VMEM((E, per_w), jnp.float32),
            pltpu.VMEM((per_w,), jnp.float32),
            pltpu.VMEM((per_w,), jnp.float32),
            pltpu.VMEM((per_w,), jnp.int32),
            pltpu.VMEM((per_w,), jnp.int32),
        ],
    )
    return f(logits_t)


def _fused_body(x_ref, w_ref, b_ref, iv_ref, vv_ref):
    lgt = lax.dot_general(
        w_ref[...], x_ref[...],
        dimension_numbers=(((1,), (1,)), ((), ())),
        preferred_element_type=jnp.float32,
    ) + b_ref[...]
    tb = lgt.shape[1]
    m = jnp.max(lgt, axis=0, keepdims=True)
    s = jnp.sum(jnp.exp(lgt - m), axis=0, keepdims=True)
    iota = lax.broadcasted_iota(jnp.int32, (E, tb), 0)
    idx1 = jnp.min(jnp.where(lgt == m, iota, E), axis=0, keepdims=True)
    lgt2 = jnp.where(iota == idx1, -jnp.inf, lgt)
    m2 = jnp.max(lgt2, axis=0, keepdims=True)
    idx2 = jnp.min(jnp.where(lgt2 == m2, iota, E), axis=0, keepdims=True)
    inv = 1.0 / s
    iv_ref[0] = jnp.concatenate([idx1, idx2], axis=0)
    vv_ref[0] = jnp.concatenate([inv, jnp.exp(m2 - m) * inv], axis=0)


def _fused_tc(x2d, W, b2d, tblk):
    nblk = N // tblk
    return pl.pallas_call(
        _fused_body,
        grid=(nblk,),
        in_specs=[
            pl.BlockSpec((tblk, D), lambda i: (i, 0)),
            pl.BlockSpec((E, D), lambda i: (0, 0)),
            pl.BlockSpec((E, 1), lambda i: (0, 0)),
        ],
        out_specs=[
            pl.BlockSpec((1, 2, tblk), lambda i: (i, 0, 0)),
            pl.BlockSpec((1, 2, tblk), lambda i: (i, 0, 0)),
        ],
        out_shape=[
            jax.ShapeDtypeStruct((nblk, 2, tblk), jnp.int32),
            jax.ShapeDtypeStruct((nblk, 2, tblk), jnp.float32),
        ],
    )(x2d, W, b2d)


def kernel(x, W, b):
    x2d = x.reshape(N, D)
    b2d = b.reshape(E, 1)
    iv, vv = _fused_tc(x2d, W, b2d, 1024)
    topk_idx = jnp.transpose(iv, (0, 2, 1)).reshape(B, T, 2)
    topk_vals = jnp.transpose(vv, (0, 2, 1)).reshape(B, T, 2)
    return (topk_idx, topk_vals)


# speedup vs baseline: 1.4006x; 1.0000x over previous
import functools

import jax
import jax.numpy as jnp
from jax import lax
from jax.experimental import pallas as pl
from jax.experimental.pallas import tpu as pltpu
from jax.experimental.pallas import tpu_sc as plsc

B, T, D = 4, 4096, 2048
E = 16
N = B * T
NW = 32
LANES = 16
CHUNKS = (4096, 4096, 4096, 4096)


def _logits_body(x_ref, w_ref, b_ref, out_ref):
    acc = lax.dot_general(
        w_ref[...], x_ref[...],
        dimension_numbers=(((1,), (1,)), ((), ())),
        preferred_element_type=jnp.float32,
    )
    out_ref[0] = acc + b_ref[...]


def _compute_logits_t(x2d, W, b2d, base_tok, chunk):
    per_w = chunk // NW
    base_blk = base_tok // per_w
    return pl.pallas_call(
        _logits_body,
        grid=(NW,),
        in_specs=[
            pl.BlockSpec((per_w, D), lambda i: (base_blk + i, 0)),
            pl.BlockSpec((E, D), lambda i: (0, 0)),
            pl.BlockSpec((E, 1), lambda i: (0, 0)),
        ],
        out_specs=pl.BlockSpec((1, E, per_w), lambda i: (i, 0, 0)),
        out_shape=jax.ShapeDtypeStruct((NW, E, per_w), jnp.float32),
    )(x2d, W, b2d)


def _route_body(per_w, lg_hbm, v1_hbm, v2_hbm, i1_hbm, i2_hbm,
                lg_v, v1_v, v2_v, i1_v, i2_v):
    wid = lax.axis_index("s") * 2 + lax.axis_index("c")
    pltpu.sync_copy(lg_hbm.at[wid], lg_v)

    def group(j, carry):
        base = j * LANES
        vecs = [lg_v[e, pl.ds(base, LANES)] for e in range(E)]
        m = vecs[0]
        for e in range(1, E):
            m = jnp.maximum(m, vecs[e])
        s = jnp.exp(vecs[0] - m)
        for e in range(1, E):
            s = s + jnp.exp(vecs[e] - m)
        max1 = vecs[0]
        idx1 = jnp.zeros((LANES,), jnp.int32)
        max2 = jnp.full((LANES,), -jnp.inf, jnp.float32)
        idx2 = jnp.zeros((LANES,), jnp.int32)
        for e in range(1, E):
            ve = vecs[e]
            eidx = jnp.full((LANES,), e, jnp.int32)
            gt1 = ve > max1
            gt2 = ve > max2
            max2 = jnp.where(gt1, max1, jnp.where(gt2, ve, max2))
            idx2 = jnp.where(gt1, idx1, jnp.where(gt2, eidx, idx2))
            max1 = jnp.where(gt1, ve, max1)
            idx1 = jnp.where(gt1, eidx, idx1)
        inv = 1.0 / s
        v1_v[pl.ds(base, LANES)] = inv
        v2_v[pl.ds(base, LANES)] = jnp.exp(max2 - m) * inv
        i1_v[pl.ds(base, LANES)] = idx1
        i2_v[pl.ds(base, LANES)] = idx2
        return carry

    lax.fori_loop(0, per_w // LANES, group, None)

    out_base = wid * per_w
    pltpu.sync_copy(v1_v, v1_hbm.at[pl.ds(out_base, per_w)])
    pltpu.sync_copy(v2_v, v2_hbm.at[pl.ds(out_base, per_w)])
    pltpu.sync_copy(i1_v, i1_hbm.at[pl.ds(out_base, per_w)])
    pltpu.sync_copy(i2_v, i2_hbm.at[pl.ds(out_base, per_w)])


def _route_topk(logits_t, chunk):
    per_w = chunk // NW
    mesh = plsc.VectorSubcoreMesh(core_axis_name="c", subcore_axis_name="s")
    f = pl.kernel(
        functools.partial(_route_body, per_w),
        out_type=[
            jax.ShapeDtypeStruct((chunk,), jnp.float32),
            jax.ShapeDtypeStruct((chunk,), jnp.float32),
            jax.ShapeDtypeStruct((chunk,), jnp.int32),
            jax.ShapeDtypeStruct((chunk,), jnp.int32),
        ],
        mesh=mesh,
        scratch_types=[
            pltpu.VMEM((E, per_w), jnp.float32),
            pltpu.VMEM((per_w,), jnp.float32),
            pltpu.VMEM((per_w,), jnp.float32),
            pltpu.VMEM((per_w,), jnp.int32),
            pltpu.VMEM((per_w,), jnp.int32),
        ],
    )
    return f(logits_t)


def _fused_body(xa_ref, xb_ref, w_ref, b_ref, iv_ref, vv_ref):
    half = D // 2
    lgt = lax.dot_general(
        w_ref[:, :half], xa_ref[...],
        dimension_numbers=(((1,), (1,)), ((), ())),
        preferred_element_type=jnp.float32,
    ) + lax.dot_general(
        w_ref[:, half:], xb_ref[...],
        dimension_numbers=(((1,), (1,)), ((), ())),
        preferred_element_type=jnp.float32,
    ) + b_ref[...]
    tb = lgt.shape[1]
    m = jnp.max(lgt, axis=0, keepdims=True)
    s = jnp.sum(jnp.exp(lgt - m), axis=0, keepdims=True)
    iota = lax.broadcasted_iota(jnp.int32, (E, tb), 0)
    idx1 = jnp.min(jnp.where(lgt == m, iota, E), axis=0, keepdims=True)
    lgt2 = jnp.where(iota == idx1, -jnp.inf, lgt)
    m2 = jnp.max(lgt2, axis=0, keepdims=True)
    idx2 = jnp.min(jnp.where(lgt2 == m2, iota, E), axis=0, keepdims=True)
    inv = 1.0 / s
    iv_ref[0] = jnp.concatenate([idx1, idx2], axis=0)
    vv_ref[0] = jnp.concatenate([inv, jnp.exp(m2 - m) * inv], axis=0)


def _fused_tc(x2d, W, b2d, tblk):
    nblk = N // tblk
    return pl.pallas_call(
        _fused_body,
        grid=(nblk,),
        in_specs=[
            pl.BlockSpec((tblk, D // 2), lambda i: (i, 0)),
            pl.BlockSpec((tblk, D // 2), lambda i: (i, 1)),
            pl.BlockSpec((E, D), lambda i: (0, 0)),
            pl.BlockSpec((E, 1), lambda i: (0, 0)),
        ],
        out_specs=[
            pl.BlockSpec((1, 2, tblk), lambda i: (i, 0, 0)),
            pl.BlockSpec((1, 2, tblk), lambda i: (i, 0, 0)),
        ],
        out_shape=[
            jax.ShapeDtypeStruct((nblk, 2, tblk), jnp.int32),
            jax.ShapeDtypeStruct((nblk, 2, tblk), jnp.float32),
        ],
    )(x2d, x2d, W, b2d)


def kernel(x, W, b):
    x2d = x.reshape(N, D)
    b2d = b.reshape(E, 1)
    iv, vv = _fused_tc(x2d, W, b2d, 1024)
    topk_idx = jnp.transpose(iv, (0, 2, 1)).reshape(B, T, 2)
    topk_vals = jnp.transpose(vv, (0, 2, 1)).reshape(B, T, 2)
    return (topk_idx, topk_vals)


# speedup vs baseline: 1.4117x; 1.0079x over previous
import jax
import jax.numpy as jnp
from jax import lax
from jax.experimental import pallas as pl

B, T, D = 4, 4096, 2048
E = 16
N = B * T
TBLK = 1024
NBLK = N // TBLK


def _fused_body(x_ref, w_ref, b_ref, iv_ref, vv_ref):
    lgt = lax.dot_general(
        w_ref[...], x_ref[...],
        dimension_numbers=(((1,), (1,)), ((), ())),
        preferred_element_type=jnp.float32,
    ) + b_ref[...]
    m = jnp.max(lgt, axis=0, keepdims=True)
    s = jnp.sum(jnp.exp(lgt - m), axis=0, keepdims=True)
    iota = lax.broadcasted_iota(jnp.int32, (E, TBLK), 0)
    idx1 = jnp.min(jnp.where(lgt == m, iota, E), axis=0, keepdims=True)
    lgt2 = jnp.where(iota == idx1, -jnp.inf, lgt)
    m2 = jnp.max(lgt2, axis=0, keepdims=True)
    idx2 = jnp.min(jnp.where(lgt2 == m2, iota, E), axis=0, keepdims=True)
    inv = 1.0 / s
    iv_ref[0] = jnp.concatenate([idx1, idx2], axis=0)
    vv_ref[0] = jnp.concatenate([inv, jnp.exp(m2 - m) * inv], axis=0)


def kernel(x, W, b):
    x2d = x.reshape(N, D)
    iv, vv = pl.pallas_call(
        _fused_body,
        grid=(NBLK,),
        in_specs=[
            pl.BlockSpec((TBLK, D), lambda i: (i, 0)),
            pl.BlockSpec((E, D), lambda i: (0, 0)),
            pl.BlockSpec((E, 1), lambda i: (0, 0)),
        ],
        out_specs=[
            pl.BlockSpec((1, 2, TBLK), lambda i: (i, 0, 0)),
            pl.BlockSpec((1, 2, TBLK), lambda i: (i, 0, 0)),
        ],
        out_shape=[
            jax.ShapeDtypeStruct((NBLK, 2, TBLK), jnp.int32),
            jax.ShapeDtypeStruct((NBLK, 2, TBLK), jnp.float32),
        ],
    )(x2d, W, b.reshape(E, 1))
    topk_idx = jnp.transpose(iv, (0, 2, 1)).reshape(B, T, 2)
    topk_vals = jnp.transpose(vv, (0, 2, 1)).reshape(B, T, 2)
    return (topk_idx, topk_vals)
